# Initial kernel scaffold; baseline (speedup 1.0000x reference)
import jax
import jax.numpy as jnp
from jax.experimental import pallas as pl


def kernel(x, edge_index, edge_weight, W_in, b_in, W1, b1, W2, b2, W_out, b_out):
    raise NotImplementedError("write your pallas kernel here")


# speedup vs baseline: 5.1892x; 5.1892x over previous
import functools
import jax
import jax.numpy as jnp
from jax import lax
from jax.experimental import pallas as pl
from jax.experimental.pallas import tpu as pltpu
from jax.experimental.pallas import tpu_sc as plsc

N = 10000
E = 320000
D = 128
H = 128
C = 40

NC, NS, L = 2, 16, 16
NW = NC * NS
FPW = H // NW
EC = 2048
NCH = 157
EPAD = NCH * EC
EPW = EPAD // NW

_mesh = plsc.VectorSubcoreMesh(core_axis_name="c", subcore_axis_name="s")
_sc_params = pltpu.CompilerParams(needs_layout_passes=False)


@functools.partial(
    pl.kernel,
    out_type=jax.ShapeDtypeStruct((NW * N,), jnp.float32),
    mesh=_mesh,
    compiler_params=_sc_params,
    scratch_types=[
        pltpu.VMEM((EPW,), jnp.int32),
        pltpu.VMEM((EPW,), jnp.float32),
        pltpu.VMEM((N,), jnp.float32),
    ],
)
def _deg_sc(dst_hbm, ew_hbm, out_hbm, dst_v, ew_v, deg_v):
    cid = lax.axis_index("c")
    sid = lax.axis_index("s")
    wid = sid * NC + cid
    zeros16 = jnp.zeros((L,), jnp.float32)

    @pl.loop(0, N // L)
    def _zero(i):
        deg_v[pl.ds(i * L, L)] = zeros16

    pltpu.sync_copy(dst_hbm.at[pl.ds(wid * EPW, EPW)], dst_v)
    pltpu.sync_copy(ew_hbm.at[pl.ds(wid * EPW, EPW)], ew_v)

    @pl.loop(0, EPW // L)
    def _grp(g):
        sl = pl.ds(g * L, L)
        plsc.addupdate_scatter(deg_v, [dst_v[sl]], ew_v[sl])

    pltpu.sync_copy(deg_v, out_hbm.at[pl.ds(wid * N, N)])


@functools.partial(
    pl.kernel,
    out_type=jax.ShapeDtypeStruct((H * N,), jnp.float32),
    mesh=_mesh,
    compiler_params=_sc_params,
    scratch_types=(
        [pltpu.VMEM((N,), jnp.float32) for _ in range(FPW)]
        + [pltpu.VMEM((N,), jnp.float32) for _ in range(FPW)]
        + [pltpu.VMEM((EC,), jnp.int32),
           pltpu.VMEM((EC,), jnp.int32),
           pltpu.VMEM((EC,), jnp.float32)]
    ),
)
def _agg_sc(gt_hbm, src_hbm, dst_hbm, ew_hbm, out_hbm, *scratch):
    feat = scratch[:FPW]
    acc = scratch[FPW:2 * FPW]
    src_v, dst_v, ew_v = scratch[2 * FPW:]
    cid = lax.axis_index("c")
    sid = lax.axis_index("s")
    wid = sid * NC + cid
    zeros16 = jnp.zeros((L,), jnp.float32)

    @pl.loop(0, N // L)
    def _zero(i):
        sl = pl.ds(i * L, L)
        for f in range(FPW):
            acc[f][sl] = zeros16

    for f in range(FPW):
        pltpu.sync_copy(gt_hbm.at[pl.ds((wid * FPW + f) * N, N)], feat[f])

    @pl.loop(0, NCH)
    def _chunk(c):
        pltpu.sync_copy(src_hbm.at[pl.ds(c * EC, EC)], src_v)
        pltpu.sync_copy(dst_hbm.at[pl.ds(c * EC, EC)], dst_v)
        pltpu.sync_copy(ew_hbm.at[pl.ds(c * EC, EC)], ew_v)

        @pl.loop(0, EC // L)
        def _grp(g):
            sl = pl.ds(g * L, L)
            si = src_v[sl]
            di = dst_v[sl]
            w = ew_v[sl]
            for f in range(FPW):
                vals = plsc.load_gather(feat[f], [si]) * w
                plsc.addupdate_scatter(acc[f], [di], vals)

    for f in range(FPW):
        pltpu.sync_copy(acc[f], out_hbm.at[pl.ds((wid * FPW + f) * N, N)])


BN = 1000


def _dinv_from(degp):
    deg = jnp.sum(degp, axis=0) + 1.0
    return (1.0 / jnp.sqrt(deg))[:, None]


def _tc_a_body(x_ref, win_ref, bin_ref, w1_ref, degp_ref, g1t_ref):
    dinv = _dinv_from(degp_ref[...])
    h = jax.nn.relu(
        jnp.dot(x_ref[...], win_ref[...], preferred_element_type=jnp.float32)
        + bin_ref[...])
    z = jnp.dot(h, w1_ref[...], preferred_element_type=jnp.float32)
    g1t_ref[...] = (z * dinv).T


def _tc_b_body(aggt_ref, gt_ref, degp_ref, b_ref, w_ref, gout_ref):
    dinv = _dinv_from(degp_ref[...])
    s = (aggt_ref[...] + gt_ref[...]).T
    h = jax.nn.relu(s * dinv + b_ref[...])
    gout_ref[...] = (
        jnp.dot(h, w_ref[...], preferred_element_type=jnp.float32) * dinv).T


def _tc_c_body(aggt_ref, gt_ref, degp_ref, b_ref, wout_ref, bout_ref, out_ref):
    dinv = _dinv_from(degp_ref[...])
    s = (aggt_ref[...] + gt_ref[...]).T
    h = jax.nn.relu(s * dinv + b_ref[...])
    out_ref[...] = (
        jnp.dot(h, wout_ref[...], preferred_element_type=jnp.float32)
        + bout_ref[...])


_tc_a = pl.pallas_call(
    _tc_a_body, out_shape=jax.ShapeDtypeStruct((H, N), jnp.float32))

_tc_b = pl.pallas_call(
    _tc_b_body, out_shape=jax.ShapeDtypeStruct((H, N), jnp.float32))

_tc_c = pl.pallas_call(
    _tc_c_body, out_shape=jax.ShapeDtypeStruct((N, C), jnp.float32))


@jax.jit
def kernel(x, edge_index, edge_weight, W_in, b_in, W1, b1, W2, b2, W_out, b_out):
    pad = EPAD - E
    zi = jnp.zeros((pad,), edge_index.dtype)
    src_p = jnp.concatenate([edge_index[0], zi])
    dst_p = jnp.concatenate([edge_index[1], zi])
    ew_p = jnp.concatenate([edge_weight, jnp.zeros((pad,), edge_weight.dtype)])

    degp = _deg_sc(dst_p, ew_p).reshape(NW, N)
    g1t = _tc_a(x, W_in, b_in.reshape(1, H), W1, degp)
    agg1t = _agg_sc(g1t.reshape(H * N), src_p, dst_p, ew_p).reshape(H, N)
    g2t = _tc_b(agg1t, g1t, degp, b1.reshape(1, H), W2)
    agg2t = _agg_sc(g2t.reshape(H * N), src_p, dst_p, ew_p).reshape(H, N)
    return _tc_c(agg2t, g2t, degp, b2.reshape(1, H), W_out, b_out.reshape(1, C))


# speedup vs baseline: 7.4486x; 1.4354x over previous
import functools
import jax
import jax.numpy as jnp
from jax import lax
from jax.experimental import pallas as pl
from jax.experimental.pallas import tpu as pltpu
from jax.experimental.pallas import tpu_sc as plsc

N = 10000
E = 320000
D = 128
H = 128
C = 40

NC, NS, L = 2, 16, 16
NW = NC * NS
FPW = H // NW
EC = 2048
NCH = 158
EPAD = NCH * EC
EPW = EPAD // NW

_mesh = plsc.VectorSubcoreMesh(core_axis_name="c", subcore_axis_name="s")
_sc_params = pltpu.CompilerParams(needs_layout_passes=False)


@functools.partial(
    pl.kernel,
    out_type=jax.ShapeDtypeStruct((NW * N,), jnp.float32),
    mesh=_mesh,
    compiler_params=_sc_params,
    scratch_types=[
        pltpu.VMEM((EPW,), jnp.int32),
        pltpu.VMEM((EPW,), jnp.float32),
        pltpu.VMEM((N,), jnp.float32),
    ],
)
def _deg_sc(dst_hbm, ew_hbm, out_hbm, dst_v, ew_v, deg_v):
    cid = lax.axis_index("c")
    sid = lax.axis_index("s")
    wid = sid * NC + cid
    zeros16 = jnp.zeros((L,), jnp.float32)

    @pl.loop(0, N // L)
    def _zero(i):
        deg_v[pl.ds(i * L, L)] = zeros16

    pltpu.sync_copy(dst_hbm.at[pl.ds(wid * EPW, EPW)], dst_v)
    pltpu.sync_copy(ew_hbm.at[pl.ds(wid * EPW, EPW)], ew_v)

    @pl.loop(0, EPW // L)
    def _grp(g):
        sl = pl.ds(g * L, L)
        plsc.addupdate_scatter(deg_v, [dst_v[sl]], ew_v[sl])

    pltpu.sync_copy(deg_v, out_hbm.at[pl.ds(wid * N, N)])


@functools.partial(
    pl.kernel,
    out_type=jax.ShapeDtypeStruct((H * N,), jnp.float32),
    mesh=_mesh,
    compiler_params=_sc_params,
    scratch_types=(
        [pltpu.VMEM((N,), jnp.float32) for _ in range(FPW)]
        + [pltpu.VMEM((N,), jnp.float32) for _ in range(FPW)]
        + [pltpu.VMEM((EC,), jnp.int32),
           pltpu.VMEM((EC,), jnp.int32),
           pltpu.VMEM((EC,), jnp.float32),
           pltpu.VMEM((EC,), jnp.int32),
           pltpu.VMEM((EC,), jnp.int32),
           pltpu.VMEM((EC,), jnp.float32),
           pltpu.SemaphoreType.DMA,
           pltpu.SemaphoreType.DMA]
    ),
)
def _agg_sc(gt_hbm, src_hbm, dst_hbm, ew_hbm, out_hbm, *scratch):
    feat = scratch[:FPW]
    acc = scratch[FPW:2 * FPW]
    buf_a = scratch[2 * FPW:2 * FPW + 3]
    buf_b = scratch[2 * FPW + 3:2 * FPW + 6]
    sem_a, sem_b = scratch[2 * FPW + 6:]
    cid = lax.axis_index("c")
    sid = lax.axis_index("s")
    wid = sid * NC + cid
    zeros16 = jnp.zeros((L,), jnp.float32)
    hbm3 = (src_hbm, dst_hbm, ew_hbm)

    def start(c, bufs, sem):
        for h, b in zip(hbm3, bufs):
            pltpu.async_copy(h.at[pl.ds(c * EC, EC)], b, sem)

    def drain(bufs, sem):
        for h, b in zip(hbm3, bufs):
            pltpu.make_async_copy(h.at[pl.ds(0, EC)], b, sem).wait()

    def compute(bufs):
        src_v, dst_v, ew_v = bufs

        @pl.loop(0, EC // L, unroll=8)
        def _grp(g):
            sl = pl.ds(g * L, L)
            si = src_v[sl]
            di = dst_v[sl]
            w = ew_v[sl]
            for f in range(FPW):
                vals = plsc.load_gather(feat[f], [si]) * w
                plsc.addupdate_scatter(acc[f], [di], vals)

    @pl.loop(0, N // L)
    def _zero(i):
        sl = pl.ds(i * L, L)
        for f in range(FPW):
            acc[f][sl] = zeros16

    start(0, buf_a, sem_a)
    for f in range(FPW):
        pltpu.sync_copy(gt_hbm.at[pl.ds((wid * FPW + f) * N, N)], feat[f])

    @pl.loop(0, NCH // 2)
    def _pair(p):
        c0 = 2 * p
        drain(buf_a, sem_a)
        start(c0 + 1, buf_b, sem_b)
        compute(buf_a)
        drain(buf_b, sem_b)

        @pl.when(c0 + 2 < NCH)
        def _prefetch():
            start(c0 + 2, buf_a, sem_a)

        compute(buf_b)

    for f in range(FPW):
        pltpu.sync_copy(acc[f], out_hbm.at[pl.ds((wid * FPW + f) * N, N)])


BN = 1000


def _dinv_from(degp):
    deg = jnp.sum(degp, axis=0) + 1.0
    return (1.0 / jnp.sqrt(deg))[:, None]


def _tc_a_body(x_ref, win_ref, bin_ref, w1_ref, degp_ref, g1t_ref):
    dinv = _dinv_from(degp_ref[...])
    h = jax.nn.relu(
        jnp.dot(x_ref[...], win_ref[...], preferred_element_type=jnp.float32)
        + bin_ref[...])
    z = jnp.dot(h, w1_ref[...], preferred_element_type=jnp.float32)
    g1t_ref[...] = (z * dinv).T


def _tc_b_body(aggt_ref, gt_ref, degp_ref, b_ref, w_ref, gout_ref):
    dinv = _dinv_from(degp_ref[...])
    s = (aggt_ref[...] + gt_ref[...]).T
    h = jax.nn.relu(s * dinv + b_ref[...])
    gout_ref[...] = (
        jnp.dot(h, w_ref[...], preferred_element_type=jnp.float32) * dinv).T


def _tc_c_body(aggt_ref, gt_ref, degp_ref, b_ref, wout_ref, bout_ref, out_ref):
    dinv = _dinv_from(degp_ref[...])
    s = (aggt_ref[...] + gt_ref[...]).T
    h = jax.nn.relu(s * dinv + b_ref[...])
    out_ref[...] = (
        jnp.dot(h, wout_ref[...], preferred_element_type=jnp.float32)
        + bout_ref[...])


_tc_a = pl.pallas_call(
    _tc_a_body, out_shape=jax.ShapeDtypeStruct((H, N), jnp.float32))

_tc_b = pl.pallas_call(
    _tc_b_body, out_shape=jax.ShapeDtypeStruct((H, N), jnp.float32))

_tc_c = pl.pallas_call(
    _tc_c_body, out_shape=jax.ShapeDtypeStruct((N, C), jnp.float32))


@jax.jit
def kernel(x, edge_index, edge_weight, W_in, b_in, W1, b1, W2, b2, W_out, b_out):
    pad = EPAD - E
    zi = jnp.zeros((pad,), edge_index.dtype)
    src_p = jnp.concatenate([edge_index[0], zi])
    dst_p = jnp.concatenate([edge_index[1], zi])
    ew_p = jnp.concatenate([edge_weight, jnp.zeros((pad,), edge_weight.dtype)])

    degp = _deg_sc(dst_p, ew_p).reshape(NW, N)
    g1t = _tc_a(x, W_in, b_in.reshape(1, H), W1, degp)
    agg1t = _agg_sc(g1t.reshape(H * N), src_p, dst_p, ew_p).reshape(H, N)
    g2t = _tc_b(agg1t, g1t, degp, b1.reshape(1, H), W2)
    agg2t = _agg_sc(g2t.reshape(H * N), src_p, dst_p, ew_p).reshape(H, N)
    return _tc_c(agg2t, g2t, degp, b2.reshape(1, H), W_out, b_out.reshape(1, C))


# speedup vs baseline: 7.4508x; 1.0003x over previous
import functools
import jax
import jax.numpy as jnp
from jax import lax
from jax.experimental import pallas as pl
from jax.experimental.pallas import tpu as pltpu
from jax.experimental.pallas import tpu_sc as plsc

N = 10000
E = 320000
D = 128
H = 128
C = 40

NC, NS, L = 2, 16, 16
NW = NC * NS
FPW = H // NW
EC = 4096
NCH = 79
EPAD = NCH * EC
EPW = EPAD // NW

_mesh = plsc.VectorSubcoreMesh(core_axis_name="c", subcore_axis_name="s")
_sc_params = pltpu.CompilerParams(needs_layout_passes=False)


@functools.partial(
    pl.kernel,
    out_type=jax.ShapeDtypeStruct((NW * N,), jnp.float32),
    mesh=_mesh,
    compiler_params=_sc_params,
    scratch_types=[
        pltpu.VMEM((EPW,), jnp.int32),
        pltpu.VMEM((EPW,), jnp.float32),
        pltpu.VMEM((N,), jnp.float32),
    ],
)
def _deg_sc(dst_hbm, ew_hbm, out_hbm, dst_v, ew_v, deg_v):
    cid = lax.axis_index("c")
    sid = lax.axis_index("s")
    wid = sid * NC + cid
    zeros16 = jnp.zeros((L,), jnp.float32)

    @pl.loop(0, N // L)
    def _zero(i):
        deg_v[pl.ds(i * L, L)] = zeros16

    pltpu.sync_copy(dst_hbm.at[pl.ds(wid * EPW, EPW)], dst_v)
    pltpu.sync_copy(ew_hbm.at[pl.ds(wid * EPW, EPW)], ew_v)

    @pl.loop(0, EPW // L)
    def _grp(g):
        sl = pl.ds(g * L, L)
        plsc.addupdate_scatter(deg_v, [dst_v[sl]], ew_v[sl])

    pltpu.sync_copy(deg_v, out_hbm.at[pl.ds(wid * N, N)])


@functools.partial(
    pl.kernel,
    out_type=jax.ShapeDtypeStruct((H * N,), jnp.float32),
    mesh=_mesh,
    compiler_params=_sc_params,
    scratch_types=(
        [pltpu.VMEM((N,), jnp.float32) for _ in range(FPW)]
        + [pltpu.VMEM((N,), jnp.float32) for _ in range(FPW)]
        + [pltpu.VMEM((EC,), jnp.int32),
           pltpu.VMEM((EC,), jnp.int32),
           pltpu.VMEM((EC,), jnp.float32),
           pltpu.VMEM((EC,), jnp.int32),
           pltpu.VMEM((EC,), jnp.int32),
           pltpu.VMEM((EC,), jnp.float32),
           pltpu.SemaphoreType.DMA,
           pltpu.SemaphoreType.DMA]
    ),
)
def _agg_sc(gt_hbm, src_hbm, dst_hbm, ew_hbm, out_hbm, *scratch):
    feat = scratch[:FPW]
    acc = scratch[FPW:2 * FPW]
    buf_a = scratch[2 * FPW:2 * FPW + 3]
    buf_b = scratch[2 * FPW + 3:2 * FPW + 6]
    sem_a, sem_b = scratch[2 * FPW + 6:]
    cid = lax.axis_index("c")
    sid = lax.axis_index("s")
    wid = sid * NC + cid
    zeros16 = jnp.zeros((L,), jnp.float32)
    hbm3 = (src_hbm, dst_hbm, ew_hbm)

    def start(c, bufs, sem):
        for h, b in zip(hbm3, bufs):
            pltpu.async_copy(h.at[pl.ds(c * EC, EC)], b, sem)

    def drain(bufs, sem):
        for h, b in zip(hbm3, bufs):
            pltpu.make_async_copy(h.at[pl.ds(0, EC)], b, sem).wait()

    def compute(bufs):
        src_v, dst_v, ew_v = bufs

        @pl.loop(0, EC // L, unroll=16)
        def _grp(g):
            sl = pl.ds(g * L, L)
            si = src_v[sl]
            di = dst_v[sl]
            w = ew_v[sl]
            for f in range(FPW):
                vals = plsc.load_gather(feat[f], [si]) * w
                plsc.addupdate_scatter(acc[f], [di], vals)

    @pl.loop(0, N // L)
    def _zero(i):
        sl = pl.ds(i * L, L)
        for f in range(FPW):
            acc[f][sl] = zeros16

    start(0, buf_a, sem_a)
    for f in range(FPW):
        pltpu.sync_copy(gt_hbm.at[pl.ds((wid * FPW + f) * N, N)], feat[f])

    @pl.loop(0, NCH // 2)
    def _pair(p):
        c0 = 2 * p
        drain(buf_a, sem_a)
        start(c0 + 1, buf_b, sem_b)
        compute(buf_a)
        drain(buf_b, sem_b)

        @pl.when(c0 + 2 < NCH)
        def _prefetch():
            start(c0 + 2, buf_a, sem_a)

        compute(buf_b)

    drain(buf_a, sem_a)
    compute(buf_a)

    for f in range(FPW):
        pltpu.sync_copy(acc[f], out_hbm.at[pl.ds((wid * FPW + f) * N, N)])


BN = 1000


def _dinv_from(degp):
    deg = jnp.sum(degp, axis=0) + 1.0
    return (1.0 / jnp.sqrt(deg))[:, None]


def _tc_a_body(x_ref, win_ref, bin_ref, w1_ref, degp_ref, g1t_ref):
    dinv = _dinv_from(degp_ref[...])
    h = jax.nn.relu(
        jnp.dot(x_ref[...], win_ref[...], preferred_element_type=jnp.float32)
        + bin_ref[...])
    z = jnp.dot(h, w1_ref[...], preferred_element_type=jnp.float32)
    g1t_ref[...] = (z * dinv).T


def _tc_b_body(aggt_ref, gt_ref, degp_ref, b_ref, w_ref, gout_ref):
    dinv = _dinv_from(degp_ref[...])
    s = (aggt_ref[...] + gt_ref[...]).T
    h = jax.nn.relu(s * dinv + b_ref[...])
    gout_ref[...] = (
        jnp.dot(h, w_ref[...], preferred_element_type=jnp.float32) * dinv).T


def _tc_c_body(aggt_ref, gt_ref, degp_ref, b_ref, wout_ref, bout_ref, out_ref):
    dinv = _dinv_from(degp_ref[...])
    s = (aggt_ref[...] + gt_ref[...]).T
    h = jax.nn.relu(s * dinv + b_ref[...])
    out_ref[...] = (
        jnp.dot(h, wout_ref[...], preferred_element_type=jnp.float32)
        + bout_ref[...])


_tc_a = pl.pallas_call(
    _tc_a_body, out_shape=jax.ShapeDtypeStruct((H, N), jnp.float32))

_tc_b = pl.pallas_call(
    _tc_b_body, out_shape=jax.ShapeDtypeStruct((H, N), jnp.float32))

_tc_c = pl.pallas_call(
    _tc_c_body, out_shape=jax.ShapeDtypeStruct((N, C), jnp.float32))


@jax.jit
def kernel(x, edge_index, edge_weight, W_in, b_in, W1, b1, W2, b2, W_out, b_out):
    pad = EPAD - E
    zi = jnp.zeros((pad,), edge_index.dtype)
    src_p = jnp.concatenate([edge_index[0], zi])
    dst_p = jnp.concatenate([edge_index[1], zi])
    ew_p = jnp.concatenate([edge_weight, jnp.zeros((pad,), edge_weight.dtype)])

    degp = _deg_sc(dst_p, ew_p).reshape(NW, N)
    g1t = _tc_a(x, W_in, b_in.reshape(1, H), W1, degp)
    agg1t = _agg_sc(g1t.reshape(H * N), src_p, dst_p, ew_p).reshape(H, N)
    g2t = _tc_b(agg1t, g1t, degp, b1.reshape(1, H), W2)
    agg2t = _agg_sc(g2t.reshape(H * N), src_p, dst_p, ew_p).reshape(H, N)
    return _tc_c(agg2t, g2t, degp, b2.reshape(1, H), W_out, b_out.reshape(1, C))
